# Initial kernel scaffold; baseline (speedup 1.0000x reference)
import jax
import jax.numpy as jnp
from jax.experimental import pallas as pl


def kernel(x, edge_index, edge_attr, batch, lin0_w, lin0_b, conv_w, lin1_w, lin1_b):
    raise NotImplementedError("write your pallas kernel here")


# speedup vs baseline: 8.9390x; 8.9390x over previous
import functools
import math

import jax
import jax.numpy as jnp
from jax import lax
from jax.experimental import pallas as pl
from jax.experimental.pallas import tpu as pltpu
from jax.experimental.pallas import tpu_sc as plsc

NUM_LAYERS = 4
ALPHA = 0.1
THETA = 0.5
NUM_GRAPHS = 64

_NC = 2
_NS = 16
_NW = _NC * _NS
_CH = 128
_BT = 256


def _sc_mesh():
    return plsc.VectorSubcoreMesh(core_axis_name="c", subcore_axis_name="s")


def _deg_counts_kernel(npad, kd, kb, gpad):
    @functools.partial(
        pl.kernel,
        out_type=(jax.ShapeDtypeStruct((_NC, npad), jnp.float32),
                  jax.ShapeDtypeStruct((_NC, gpad), jnp.float32)),
        mesh=_sc_mesh(),
        scratch_types=[
            pltpu.VMEM((kd, _CH), jnp.int32),
            pltpu.VMEM((kb, _CH), jnp.int32),
            pltpu.VMEM((_CH,), jnp.float32),
            pltpu.VMEM_SHARED((npad,), jnp.float32),
            pltpu.VMEM_SHARED((gpad,), jnp.float32),
        ],
    )
    def k(col_hbm, bat_hbm, zeros_hbm, deg_out, cnt_out,
          cidx, bidx, ones_v, deg_sh, cnt_sh):
        cid = lax.axis_index("c")
        sid = lax.axis_index("s")
        wid = cid * _NS + sid
        rp = npad // _NS
        pltpu.sync_copy(zeros_hbm.at[pl.ds(sid * rp, rp)],
                        deg_sh.at[pl.ds(sid * rp, rp)])

        @pl.when(sid == 0)
        def _():
            pltpu.sync_copy(zeros_hbm.at[pl.ds(0, gpad)], cnt_sh)
        pltpu.sync_copy(col_hbm.at[wid], cidx)
        pltpu.sync_copy(bat_hbm.at[wid], bidx)
        for j in range(_CH // 16):
            ones_v[pl.ds(j * 16, 16)] = jnp.ones((16,), jnp.float32)
        plsc.subcore_barrier()

        def dbody(j, c):
            pltpu.sync_copy(ones_v, deg_sh.at[cidx.at[j]], add=True)
            return c
        lax.fori_loop(0, kd, dbody, 0)

        def bbody(j, c):
            pltpu.sync_copy(ones_v, cnt_sh.at[bidx.at[j]], add=True)
            return c
        lax.fori_loop(0, kb, bbody, 0)

        plsc.subcore_barrier()
        pltpu.sync_copy(deg_sh.at[pl.ds(sid * rp, rp)],
                        deg_out.at[cid, pl.ds(sid * rp, rp)])

        @pl.when(sid == 0)
        def _():
            pltpu.sync_copy(cnt_sh, cnt_out.at[cid])

    return k


def _scatter_rows_kernel(mpad, k):
    @functools.partial(
        pl.kernel,
        out_type=jax.ShapeDtypeStruct((_NC, mpad, 128), jnp.float32),
        mesh=_sc_mesh(),
        scratch_types=[
            pltpu.VMEM((k, _CH), jnp.int32),
            pltpu.VMEM((k, _CH), jnp.int32),
            pltpu.VMEM((_CH, 128), jnp.float32),
            pltpu.VMEM_SHARED((mpad, 128), jnp.float32),
            pltpu.SemaphoreType.DMA,
        ],
    )
    def kfn(tab_hbm, ridx_hbm, cidx_hbm, zeros_hbm, out_hbm,
            ridx, cidx, rows_v, agg_sh, sem):
        cid = lax.axis_index("c")
        sid = lax.axis_index("s")
        wid = cid * _NS + sid
        rp = mpad // _NS
        pltpu.sync_copy(zeros_hbm.at[pl.ds(sid * rp, rp)],
                        agg_sh.at[pl.ds(sid * rp, rp)])
        pltpu.sync_copy(ridx_hbm.at[wid], ridx)
        pltpu.sync_copy(cidx_hbm.at[wid], cidx)
        plsc.subcore_barrier()

        def body(j, c):
            pltpu.async_copy(tab_hbm.at[ridx.at[j]], rows_v, sem).wait()
            pltpu.sync_copy(rows_v, agg_sh.at[cidx.at[j]], add=True)
            return c
        lax.fori_loop(0, k, body, 0)

        plsc.subcore_barrier()
        pltpu.sync_copy(agg_sh.at[pl.ds(sid * rp, rp)],
                        out_hbm.at[cid, pl.ds(sid * rp, rp)])

    return kfn


def _lin0_call(xp, w, b, degp, npad):
    nblk = npad // _BT

    def body(x_ref, w_ref, b_ref, deg_ref, h_ref, hs_ref, db_ref):
        d = deg_ref[0, :] + deg_ref[1, :] + 1.0
        dinv = lax.rsqrt(d)
        h = jnp.maximum(
            jnp.dot(x_ref[...], w_ref[...],
                    preferred_element_type=jnp.float32) + b_ref[...], 0.0)
        db = jnp.broadcast_to(dinv[:, None], h.shape)
        h_ref[...] = h
        hs_ref[...] = h * db
        db_ref[...] = db

    o = jax.ShapeDtypeStruct((npad, 128), jnp.float32)
    return pl.pallas_call(
        body,
        grid=(nblk,),
        in_specs=[
            pl.BlockSpec((_BT, 128), lambda i: (i, 0)),
            pl.BlockSpec((128, 128), lambda i: (0, 0)),
            pl.BlockSpec((1, 128), lambda i: (0, 0)),
            pl.BlockSpec((2, _BT), lambda i: (0, i)),
        ],
        out_specs=[pl.BlockSpec((_BT, 128), lambda i: (i, 0))] * 3,
        out_shape=[o, o, o],
    )(xp, w, b, degp)


def _layer_call(p, h, x0, db, w, beta, npad):
    nblk = npad // _BT
    a1 = 1.0 - ALPHA
    b1 = 1.0 - beta

    def body(p_ref, h_ref, x0_ref, db_ref, w_ref, hn_ref, hs_ref):
        dbv = db_ref[...]
        s = p_ref[0] + p_ref[1]
        agg = dbv * s + dbv * dbv * h_ref[...]
        out = a1 * agg + ALPHA * x0_ref[...]
        m = jnp.dot(out, w_ref[...], preferred_element_type=jnp.float32)
        hn = jnp.maximum(b1 * out + beta * m, 0.0)
        hn_ref[...] = hn
        hs_ref[...] = hn * dbv

    o = jax.ShapeDtypeStruct((npad, 128), jnp.float32)
    return pl.pallas_call(
        body,
        grid=(nblk,),
        in_specs=[
            pl.BlockSpec((2, _BT, 128), lambda i: (0, i, 0)),
            pl.BlockSpec((_BT, 128), lambda i: (i, 0)),
            pl.BlockSpec((_BT, 128), lambda i: (i, 0)),
            pl.BlockSpec((_BT, 128), lambda i: (i, 0)),
            pl.BlockSpec((128, 128), lambda i: (0, 0)),
        ],
        out_specs=[pl.BlockSpec((_BT, 128), lambda i: (i, 0))] * 2,
        out_shape=[o, o],
    )(p, h, x0, db, w)


def _final_call(pp, cntp, w, b):
    def body(pp_ref, c_ref, w_ref, b_ref, o_ref):
        cnt = c_ref[0] + c_ref[1]
        s = pp_ref[0] + pp_ref[1]
        pooled = s / jnp.maximum(cnt, 1.0)[:, None]
        res = jnp.dot(pooled, w_ref[...],
                      preferred_element_type=jnp.float32) + b_ref[...]
        o_ref[...] = res[:NUM_GRAPHS]

    return pl.pallas_call(
        body,
        out_shape=jax.ShapeDtypeStruct((NUM_GRAPHS, 128), jnp.float32),
    )(pp, cntp, w, b)


def _ceil_to(v, m):
    return -(-v // m) * m


def kernel(x, edge_index, edge_attr, batch, lin0_w, lin0_b, conv_w,
           lin1_w, lin1_b):
    n = x.shape[0]
    e = edge_index.shape[1]
    npad = _ceil_to(n + 1, 2048)
    gpad = 128

    ke = _ceil_to(e, _NW * _CH) // (_NW * _CH)
    ep = ke * _NW * _CH
    kb = _ceil_to(n, _NW * _CH) // (_NW * _CH)
    nb = kb * _NW * _CH

    row = edge_index[0]
    col = edge_index[1]
    rowr = jnp.concatenate(
        [row, jnp.zeros((ep - e,), jnp.int32)]).reshape(_NW, ke, _CH)
    colr = jnp.concatenate(
        [col, jnp.full((ep - e,), n, jnp.int32)]).reshape(_NW, ke, _CH)
    batr = jnp.concatenate(
        [batch, jnp.full((nb - n,), NUM_GRAPHS, jnp.int32)]
    ).reshape(_NW, kb, _CH)
    poolr = jnp.concatenate(
        [jnp.arange(n, dtype=jnp.int32), jnp.zeros((nb - n,), jnp.int32)]
    ).reshape(_NW, kb, _CH)

    zeros_n1 = jnp.zeros((npad,), jnp.float32)
    zeros_n2 = jnp.zeros((npad, 128), jnp.float32)
    zeros_g2 = jnp.zeros((gpad, 128), jnp.float32)
    xp = jnp.zeros((npad, 128), jnp.float32).at[:n].set(x)

    degp, cntp = _deg_counts_kernel(npad, ke, kb, gpad)(colr, batr, zeros_n1)
    h0, hs, db = _lin0_call(xp, lin0_w, lin0_b.reshape(1, 128), degp, npad)

    edge_scatter = _scatter_rows_kernel(npad, ke)
    h = h0
    for layer in range(NUM_LAYERS):
        beta = math.log(THETA / (layer + 1) + 1.0)
        p = edge_scatter(hs, rowr, colr, zeros_n2)
        h, hs = _layer_call(p, h, h0, db, conv_w[layer], beta, npad)

    pp = _scatter_rows_kernel(gpad, kb)(h, poolr, batr, zeros_g2)
    return _final_call(pp, cntp, lin1_w, lin1_b.reshape(1, 128))
